# Initial kernel scaffold; baseline (speedup 1.0000x reference)
import jax
import jax.numpy as jnp
from jax.experimental import pallas as pl


def kernel(scaled_atomic_energy, atom_type, scale, shift):
    raise NotImplementedError("write your pallas kernel here")


# speedup vs baseline: 1.0529x; 1.0529x over previous
import functools

import jax
import jax.numpy as jnp
from jax import lax
from jax.experimental import pallas as pl
from jax.experimental.pallas import tpu as pltpu
from jax.experimental.pallas import tpu_sc as plsc

L = 16
NC = 2
NS = 16
NW = NC * NS


@functools.lru_cache(maxsize=None)
def _build(n):
    vecs_per_w = -(-n // (NW * L))
    chunk = vecs_per_w * L
    last_base = n - chunk

    mesh = plsc.VectorSubcoreMesh(core_axis_name="c", subcore_axis_name="s")

    @functools.partial(
        pl.kernel,
        mesh=mesh,
        compiler_params=pltpu.CompilerParams(needs_layout_passes=False),
        out_type=jax.ShapeDtypeStruct((n,), jnp.float32),
        scratch_types=[
            pltpu.VMEM((chunk,), jnp.float32),
            pltpu.VMEM((chunk,), jnp.int32),
            pltpu.VMEM((chunk,), jnp.float32),
            pltpu.VMEM((L,), jnp.float32),
            pltpu.VMEM((L,), jnp.float32),
        ],
    )
    def rescale(x_hbm, t_hbm, scale_hbm, shift_hbm, out_hbm,
                x_v, t_v, y_v, sc_v, sh_v):
        wid = lax.axis_index("s") * NC + lax.axis_index("c")
        base = jnp.minimum(wid * chunk, last_base)
        pltpu.sync_copy(scale_hbm, sc_v)
        pltpu.sync_copy(shift_hbm, sh_v)
        pltpu.sync_copy(x_hbm.at[pl.ds(base, chunk)], x_v)
        pltpu.sync_copy(t_hbm.at[pl.ds(base, chunk)], t_v)

        def body(i, carry):
            off = i * L
            t = t_v[pl.ds(off, L)]
            x = x_v[pl.ds(off, L)]
            s = plsc.load_gather(sc_v, [t])
            b = plsc.load_gather(sh_v, [t])
            y_v[pl.ds(off, L)] = x * s + b
            return carry

        lax.fori_loop(0, vecs_per_w, body, 0)
        pltpu.sync_copy(y_v, out_hbm.at[pl.ds(base, chunk)])

    return rescale


def kernel(scaled_atomic_energy, atom_type, scale, shift):
    n = scaled_atomic_energy.shape[0]
    x = scaled_atomic_energy.reshape(n)
    t = atom_type.astype(jnp.int32)
    y = _build(n)(x, t, scale, shift)
    return y.reshape(n, 1)


# speedup vs baseline: 1.1449x; 1.0874x over previous
import functools

import jax
import jax.numpy as jnp
from jax import lax
from jax.experimental import pallas as pl
from jax.experimental.pallas import tpu as pltpu
from jax.experimental.pallas import tpu_sc as plsc

L = 16
NC = 2
NS = 16
NW = NC * NS


@functools.lru_cache(maxsize=None)
def _build(n):
    vecs_per_w = -(-n // (NW * L))
    chunk = vecs_per_w * L
    last_base = n - chunk

    mesh = plsc.VectorSubcoreMesh(core_axis_name="c", subcore_axis_name="s")

    @functools.partial(
        pl.kernel,
        mesh=mesh,
        compiler_params=pltpu.CompilerParams(needs_layout_passes=False),
        out_type=jax.ShapeDtypeStruct((n,), jnp.float32),
        scratch_types=[
            pltpu.VMEM((chunk,), jnp.float32),
            pltpu.VMEM((chunk,), jnp.int32),
            pltpu.VMEM((chunk,), jnp.float32),
            pltpu.VMEM((L,), jnp.float32),
            pltpu.VMEM((L,), jnp.float32),
            pltpu.SemaphoreType.DMA,
        ],
    )
    def rescale(x_hbm, t_hbm, scale_hbm, shift_hbm, out_hbm,
                x_v, t_v, y_v, sc_v, sh_v, sem):
        wid = lax.axis_index("s") * NC + lax.axis_index("c")
        base = jnp.minimum(wid * chunk, last_base)
        c1 = pltpu.async_copy(scale_hbm, sc_v, sem)
        c2 = pltpu.async_copy(shift_hbm, sh_v, sem)
        c3 = pltpu.async_copy(x_hbm.at[pl.ds(base, chunk)], x_v, sem)
        c4 = pltpu.async_copy(t_hbm.at[pl.ds(base, chunk)], t_v, sem)
        c1.wait()
        c2.wait()
        c3.wait()
        c4.wait()

        @plsc.parallel_loop(0, vecs_per_w, unroll=4)
        def body(i):
            off = i * L
            t = t_v[pl.ds(off, L)]
            x = x_v[pl.ds(off, L)]
            s = plsc.load_gather(sc_v, [t])
            b = plsc.load_gather(sh_v, [t])
            y_v[pl.ds(off, L)] = x * s + b

        pltpu.sync_copy(y_v, out_hbm.at[pl.ds(base, chunk)])

    return rescale


def kernel(scaled_atomic_energy, atom_type, scale, shift):
    n = scaled_atomic_energy.shape[0]
    x = scaled_atomic_energy.reshape(n)
    t = atom_type.astype(jnp.int32)
    y = _build(n)(x, t, scale, shift)
    return y.reshape(n, 1)
